# baseline (speedup 1.0000x reference)
import jax
import jax.numpy as jnp
from jax import lax
from jax.experimental import pallas as pl
from jax.experimental.pallas import tpu as pltpu
from jax.experimental.pallas import tpu_sc as plsc

VOCAB = 1_000_000
DIM = 32
BATCH = 4096
HIST = 200

NUM_CORES = 2
NUM_SUBCORES = 16
NW = NUM_CORES * NUM_SUBCORES
BLK = BATCH // NW
HCHUNK = 4
CHUNK = BLK * HCHUNK
NCHUNK = HIST // HCHUNK
NBUF = 2
L = 16
D_HI = DIM // 8


def _emb_body(table_hbm, idx_hbm, out_hbm, idx_all, idx_t, rows_v, tbuf,
              isem, gsem0, gsem1, osem0, osem1):
    gsems = (gsem0, gsem1)
    osems = (osem0, osem1)

    wid = lax.axis_index("s") * NUM_CORES + lax.axis_index("c")
    row0 = wid * BLK
    lanes = lax.iota(jnp.int32, L)

    def idx_tr(b, g):
        for hh in range(HCHUNK):
            h = g * HCHUNK + hh
            for blg in range(BLK // L):
                bl = blg * L + lanes
                v = plsc.load_gather(
                    idx_all, [bl, jnp.full((L,), 0, jnp.int32) + h])
                idx_t[b, pl.ds(hh * BLK + blg * L, L)] = v

    def gather(b):
        return pltpu.make_async_copy(
            table_hbm.at[idx_t.at[b]], rows_v.at[b], gsems[b])

    def row_tr(b):
        def hh_body(hh, carry):
            for dh in range(D_HI):
                for dl in range(8):
                    d = jnp.full((L,), dh * 8 + dl, jnp.int32)
                    for blg in range(BLK // L):
                        n = hh * BLK + blg * L + lanes
                        v = plsc.load_gather(rows_v.at[b], [n, d])
                        tbuf[b, hh, dh, dl, pl.ds(blg * L, L)] = v
            return carry

        lax.fori_loop(0, HCHUNK, hh_body, 0, unroll=False)

    def out_dmas(b, g):
        return [
            pltpu.make_async_copy(
                tbuf.at[b, hh], out_hbm.at[g * HCHUNK + hh, :, wid],
                osems[b])
            for hh in range(HCHUNK)
        ]

    pltpu.make_async_copy(
        idx_hbm.at[pl.ds(row0, BLK), :], idx_all, isem).start()
    pltpu.make_async_copy(
        idx_hbm.at[pl.ds(row0, BLK), :], idx_all, isem).wait()
    idx_tr(0, 0)
    gather(0).start()

    def step(g, b):
        bo = 1 - b

        @pl.when(g + 1 < NCHUNK)
        def _():
            idx_tr(bo, g + 1)

        gather(b).wait()

        @pl.when(g + 1 < NCHUNK)
        def _():
            gather(bo).start()

        @pl.when(g >= 2)
        def _():
            for c in out_dmas(b, g - 2):
                c.wait()

        row_tr(b)
        for c in out_dmas(b, g):
            c.start()

    def outer(i, carry):
        step(2 * i, 0)
        step(2 * i + 1, 1)
        return carry

    lax.fori_loop(0, NCHUNK // 2, outer, 0, unroll=False)

    for c in out_dmas((NCHUNK - 2) % 2, NCHUNK - 2):
        c.wait()
    for c in out_dmas((NCHUNK - 1) % 2, NCHUNK - 1):
        c.wait()


_emb = pl.kernel(
    _emb_body,
    out_type=jax.ShapeDtypeStruct((HIST, D_HI, NW, 8, BLK), jnp.float32),
    mesh=plsc.VectorSubcoreMesh(core_axis_name="c", subcore_axis_name="s"),
    scratch_types=[
        pltpu.VMEM((BLK, HIST), jnp.int32),
        pltpu.VMEM((NBUF, CHUNK), jnp.int32),
        pltpu.VMEM((NBUF, CHUNK, DIM), jnp.float32),
        pltpu.VMEM((NBUF, HCHUNK, D_HI, 8, BLK), jnp.float32),
        pltpu.SemaphoreType.DMA,
        pltpu.SemaphoreType.DMA,
        pltpu.SemaphoreType.DMA,
        pltpu.SemaphoreType.DMA,
        pltpu.SemaphoreType.DMA,
    ],
    compiler_params=pltpu.CompilerParams(
        use_tc_tiling_on_sc=False, needs_layout_passes=False),
)


def kernel(inputs, embedding_table):
    out5 = _emb(embedding_table, inputs.astype(jnp.int32))
    return out5.transpose(2, 4, 0, 1, 3).reshape(BATCH, HIST, DIM)


# speedup vs baseline: 1.1681x; 1.1681x over previous
import jax
import jax.numpy as jnp
from jax import lax
from jax.experimental import pallas as pl
from jax.experimental.pallas import tpu as pltpu
from jax.experimental.pallas import tpu_sc as plsc

VOCAB = 1_000_000
DIM = 32
BATCH = 4096
HIST = 200

NUM_CORES = 2
NUM_SUBCORES = 16
NW = NUM_CORES * NUM_SUBCORES
BLK = BATCH // NW
HCHUNK = 4
CHUNK = BLK * HCHUNK
NCHUNK = HIST // HCHUNK
NBUF = 2
L = 16
D_HI = DIM // 8


def _emb_body(table_hbm, idx_hbm, out_hbm, idx_all, idx_t, rows_v, tbuf,
              isem, gsem0, gsem1, osem0, osem1):
    gsems = (gsem0, gsem1)
    osems = (osem0, osem1)

    wid = lax.axis_index("s") * NUM_CORES + lax.axis_index("c")
    row0 = wid * BLK
    lanes = lax.iota(jnp.int32, L)

    def idx_tr(b, g):
        @plsc.parallel_loop(0, HCHUNK * (BLK // L), unroll=4)
        def _(j):
            hh = j >> 3
            bl = ((j & 7) << 4) + lanes
            h = jnp.full((L,), 0, jnp.int32) + g * HCHUNK + hh
            v = plsc.load_gather(idx_all, [bl, h])
            idx_t[b, pl.ds(j * L, L)] = v

    def gather(b):
        return pltpu.make_async_copy(
            table_hbm.at[idx_t.at[b]], rows_v.at[b], gsems[b])

    def row_tr(b):
        @plsc.parallel_loop(0, HCHUNK * DIM * (BLK // L), unroll=8)
        def _(k):
            j = k >> 3
            blg = k & 7
            d = jnp.full((L,), 0, jnp.int32) + (j & (DIM - 1))
            n = ((j >> 5) << 7) + (blg << 4) + lanes
            v = plsc.load_gather(rows_v.at[b], [n, d])
            tbuf[b, j, pl.ds(blg * L, L)] = v

    def out_dmas(b, g):
        return [
            pltpu.make_async_copy(
                tbuf.at[b, pl.ds((hh * D_HI + dh) * 8, 8), :],
                out_hbm.at[g * HCHUNK + hh, dh, wid], osems[b])
            for hh in range(HCHUNK)
            for dh in range(D_HI)
        ]

    pltpu.make_async_copy(
        idx_hbm.at[pl.ds(row0, BLK), :], idx_all, isem).start()
    pltpu.make_async_copy(
        idx_hbm.at[pl.ds(row0, BLK), :], idx_all, isem).wait()
    idx_tr(0, 0)
    gather(0).start()

    def step(g, b):
        bo = 1 - b

        @pl.when(g + 1 < NCHUNK)
        def _():
            idx_tr(bo, g + 1)

        gather(b).wait()

        @pl.when(g + 1 < NCHUNK)
        def _():
            gather(bo).start()

        @pl.when(g >= 2)
        def _():
            for c in out_dmas(b, g - 2):
                c.wait()

        row_tr(b)
        for c in out_dmas(b, g):
            c.start()

    def outer(i, carry):
        step(2 * i, 0)
        step(2 * i + 1, 1)
        return carry

    lax.fori_loop(0, NCHUNK // 2, outer, 0, unroll=False)

    for c in out_dmas((NCHUNK - 2) % 2, NCHUNK - 2):
        c.wait()
    for c in out_dmas((NCHUNK - 1) % 2, NCHUNK - 1):
        c.wait()


_emb = pl.kernel(
    _emb_body,
    out_type=jax.ShapeDtypeStruct((HIST, D_HI, NW, 8, BLK), jnp.float32),
    mesh=plsc.VectorSubcoreMesh(core_axis_name="c", subcore_axis_name="s"),
    scratch_types=[
        pltpu.VMEM((BLK, HIST), jnp.int32),
        pltpu.VMEM((NBUF, CHUNK), jnp.int32),
        pltpu.VMEM((NBUF, CHUNK, DIM), jnp.float32),
        pltpu.VMEM((NBUF, HCHUNK * DIM, BLK), jnp.float32),
        pltpu.SemaphoreType.DMA,
        pltpu.SemaphoreType.DMA,
        pltpu.SemaphoreType.DMA,
        pltpu.SemaphoreType.DMA,
        pltpu.SemaphoreType.DMA,
    ],
    compiler_params=pltpu.CompilerParams(
        use_tc_tiling_on_sc=False, needs_layout_passes=False),
)


def kernel(inputs, embedding_table):
    out5 = _emb(embedding_table, inputs.astype(jnp.int32))
    return out5.transpose(2, 4, 0, 1, 3).reshape(BATCH, HIST, DIM)


# speedup vs baseline: 1.9200x; 1.6437x over previous
import jax
import jax.numpy as jnp
from jax import lax
from jax.experimental import pallas as pl
from jax.experimental.pallas import tpu as pltpu
from jax.experimental.pallas import tpu_sc as plsc

VOCAB = 1_000_000
DIM = 32
BATCH = 4096
HIST = 200

NUM_CORES = 2
NUM_SUBCORES = 16
NW = NUM_CORES * NUM_SUBCORES
BLK = BATCH // NW
HCHUNK = 4
CHUNK = BLK * HCHUNK
NCHUNK = HIST // HCHUNK
NBUF = 2
L = 16
D_HI = DIM // 8


def _emb_body(table_hbm, idx_hbm, out_hbm, idx_all, idx_t, rows_v, tbuf,
              isem, gsem0, gsem1, osem0, osem1):
    gsems = (gsem0, gsem1)
    osems = (osem0, osem1)

    wid = lax.axis_index("s") * NUM_CORES + lax.axis_index("c")
    row0 = wid * BLK
    lanes = lax.iota(jnp.int32, L)

    def idx_tr(b, g):
        @plsc.parallel_loop(0, HCHUNK * (BLK // L), unroll=4)
        def _(j):
            hh = j >> 3
            bl = ((j & 7) << 4) + lanes
            h = jnp.full((L,), 0, jnp.int32) + g * HCHUNK + hh
            v = plsc.load_gather(idx_all, [bl, h])
            idx_t[b, pl.ds(j * L, L)] = v

    def gather(b):
        return pltpu.make_async_copy(
            table_hbm.at[idx_t.at[b]], rows_v.at[b], gsems[b])

    def row_tr(b):
        @plsc.parallel_loop(0, HCHUNK * (DIM // L) * (BLK // L), unroll=2)
        def _(j):
            hh = j >> 4
            dg = (j >> 3) & 1
            blg = j & 7
            n_vec = (hh << 7) + (blg << 4) + lanes
            col_vec = (blg << 4) + lanes
            for r in range(L):
                diag = (lanes + r) & (L - 1)
                d_vec = (dg << 4) + diag
                v = plsc.load_gather(rows_v.at[b], [n_vec, d_vec])
                row_vec = (hh << 5) + d_vec
                plsc.store_scatter(tbuf.at[b], [row_vec, col_vec], v)

    def out_dmas(b, g):
        return [
            pltpu.make_async_copy(
                tbuf.at[b, pl.ds((hh * D_HI + dh) * 8, 8), :],
                out_hbm.at[g * HCHUNK + hh, dh, wid], osems[b])
            for hh in range(HCHUNK)
            for dh in range(D_HI)
        ]

    pltpu.make_async_copy(
        idx_hbm.at[pl.ds(row0, BLK), :], idx_all, isem).start()
    pltpu.make_async_copy(
        idx_hbm.at[pl.ds(row0, BLK), :], idx_all, isem).wait()
    idx_tr(0, 0)
    gather(0).start()

    def step(g, b):
        bo = 1 - b

        @pl.when(g + 1 < NCHUNK)
        def _():
            idx_tr(bo, g + 1)

        gather(b).wait()

        @pl.when(g + 1 < NCHUNK)
        def _():
            gather(bo).start()

        @pl.when(g >= 2)
        def _():
            for c in out_dmas(b, g - 2):
                c.wait()

        row_tr(b)
        for c in out_dmas(b, g):
            c.start()

    def outer(i, carry):
        step(2 * i, 0)
        step(2 * i + 1, 1)
        return carry

    lax.fori_loop(0, NCHUNK // 2, outer, 0, unroll=False)

    for c in out_dmas((NCHUNK - 2) % 2, NCHUNK - 2):
        c.wait()
    for c in out_dmas((NCHUNK - 1) % 2, NCHUNK - 1):
        c.wait()


_emb = pl.kernel(
    _emb_body,
    out_type=jax.ShapeDtypeStruct((HIST, D_HI, NW, 8, BLK), jnp.float32),
    mesh=plsc.VectorSubcoreMesh(core_axis_name="c", subcore_axis_name="s"),
    scratch_types=[
        pltpu.VMEM((BLK, HIST), jnp.int32),
        pltpu.VMEM((NBUF, CHUNK), jnp.int32),
        pltpu.VMEM((NBUF, CHUNK, DIM), jnp.float32),
        pltpu.VMEM((NBUF, HCHUNK * DIM, BLK), jnp.float32),
        pltpu.SemaphoreType.DMA,
        pltpu.SemaphoreType.DMA,
        pltpu.SemaphoreType.DMA,
        pltpu.SemaphoreType.DMA,
        pltpu.SemaphoreType.DMA,
    ],
    compiler_params=pltpu.CompilerParams(
        use_tc_tiling_on_sc=False, needs_layout_passes=False),
)


def kernel(inputs, embedding_table):
    out5 = _emb(embedding_table, inputs.astype(jnp.int32))
    return out5.transpose(2, 4, 0, 1, 3).reshape(BATCH, HIST, DIM)
